# baseline (speedup 1.0000x reference)
import functools

import jax
import jax.numpy as jnp
from jax import lax
from jax.experimental import pallas as pl
from jax.experimental.pallas import tpu as pltpu
from jax.experimental.pallas import tpu_sc as plsc

_D = 32
_BB = 256
_HG = 8
_NS = 4


@functools.cache
def _build(batch: int, hist: int, vocab: int):
    info = plsc.get_sparse_core_info()
    nc, ns, nl = info.num_cores, info.num_subcores, info.num_lanes
    nw = nc * ns
    b_per_w = batch // nw
    assert batch % nw == 0 and b_per_w % _BB == 0
    nbb = b_per_w // _BB
    n_hg_full = hist // _HG
    h_tail = hist - n_hg_full * _HG
    seg = _BB // _NS
    mesh = plsc.VectorSubcoreMesh(core_axis_name="c", subcore_axis_name="s")

    @functools.partial(
        pl.kernel,
        mesh=mesh,
        out_type=jax.ShapeDtypeStruct((hist, _D, batch), jnp.float32),
        compiler_params=pltpu.CompilerParams(needs_layout_passes=False),
        scratch_types=[
            pltpu.VMEM((_HG, _BB), jnp.int32),
            pltpu.VMEM((2, _BB), jnp.int32),
            pltpu.VMEM((2, _BB, 4 * _D), jnp.float32),
            pltpu.VMEM((2, _D, _BB), jnp.float32),
            pltpu.SemaphoreType.DMA,
            pltpu.SemaphoreType.DMA,
            pltpu.SemaphoreType.DMA,
            pltpu.SemaphoreType.DMA,
        ],
    )
    def emb(xt_hbm, tbl_hbm, out_hbm, idx_v, line_v, rows_v, obuf_v,
            gs0, gs1, os0, os1):
        wid = lax.axis_index("s") * nc + lax.axis_index("c")
        lane_iota = lax.iota(jnp.int32, nl)
        gsems = (gs0, gs1)
        osems = (os0, os1)

        def do_block(h0, n_h, b0, owaits):
            pltpu.sync_copy(xt_hbm.at[pl.ds(h0, n_h), pl.ds(b0, _BB)],
                            idx_v.at[pl.ds(0, n_h)])

            def fire(hh):
                p = hh & 1
                for k in range(_BB // nl):
                    line_v[p, pl.ds(k * nl, nl)] = (
                        idx_v[hh, pl.ds(k * nl, nl)] >> 2)
                return tuple(
                    pltpu.async_copy(
                        tbl_hbm.at[line_v.at[p, pl.ds(s * seg, seg)]],
                        rows_v.at[p, pl.ds(s * seg, seg), :], gsems[p])
                    for s in range(_NS))

            gwaits = {0: fire(0)}
            for hh in range(n_h):
                p = hh & 1
                if hh + 1 < n_h:
                    gwaits[hh + 1] = fire(hh + 1)
                for hdl in gwaits.pop(hh):
                    hdl.wait()
                if owaits[p] is not None:
                    owaits[p].wait()

                def perm_group(g, carry):
                    toks = idx_v[hh, pl.ds(g * nl, nl)]
                    col0 = (toks & 3) << 5
                    row_ids = g * nl + lane_iota
                    for d in range(_D):
                        obuf_v[p, d, pl.ds(g * nl, nl)] = plsc.load_gather(
                            rows_v.at[p], [row_ids, col0 + d])
                    return carry

                lax.fori_loop(0, _BB // nl, perm_group, 0)
                owaits[p] = pltpu.async_copy(
                    obuf_v.at[p], out_hbm.at[h0 + hh, :, pl.ds(b0, _BB)],
                    osems[p])
            return owaits

        def hg_body(hg, carry):
            h0 = pl.multiple_of(hg * _HG, _HG)
            owaits = [None, None]
            for bb in range(nbb):
                b0 = pl.multiple_of(wid * b_per_w + bb * _BB, _BB)
                owaits = do_block(h0, _HG, b0, owaits)
            for w in owaits:
                if w is not None:
                    w.wait()
            return carry

        lax.fori_loop(0, n_hg_full, hg_body, 0)
        if h_tail:
            owaits = [None, None]
            for bb in range(nbb):
                b0 = pl.multiple_of(wid * b_per_w + bb * _BB, _BB)
                owaits = do_block(n_hg_full * _HG, h_tail, b0, owaits)
            for w in owaits:
                if w is not None:
                    w.wait()

    return emb


def kernel(x, table):
    b, h = x.shape
    v, d = table.shape
    assert d == _D and v % 4 == 0
    tbl4 = table.reshape(v // 4, 4 * _D)
    out_t = _build(b, h, v)(x.T, tbl4)
    return out_t.transpose(2, 0, 1)


# speedup vs baseline: 1.0613x; 1.0613x over previous
import functools

import jax
import jax.numpy as jnp
from jax import lax
from jax.experimental import pallas as pl
from jax.experimental.pallas import tpu as pltpu
from jax.experimental.pallas import tpu_sc as plsc

_D = 32
_BB = 256
_HG = 8
_NS = 4


@functools.cache
def _build(batch: int, hist: int, vocab: int):
    info = plsc.get_sparse_core_info()
    nc, ns, nl = info.num_cores, info.num_subcores, info.num_lanes
    nw = nc * ns
    b_per_w = batch // nw
    assert batch % nw == 0 and b_per_w % _BB == 0
    nbb = b_per_w // _BB
    n_hg_full = hist // _HG
    h_tail = hist - n_hg_full * _HG
    seg = _BB // _NS
    mesh = plsc.VectorSubcoreMesh(core_axis_name="c", subcore_axis_name="s")

    @functools.partial(
        pl.kernel,
        mesh=mesh,
        out_type=jax.ShapeDtypeStruct((hist, _D, batch), jnp.float32),
        compiler_params=pltpu.CompilerParams(needs_layout_passes=False),
        scratch_types=[
            pltpu.VMEM((_HG, _BB + 16), jnp.int32),
            pltpu.VMEM((2, _BB), jnp.int32),
            pltpu.VMEM((2, _BB, 4 * _D), jnp.float32),
            pltpu.VMEM((2, _D, _BB + 1), jnp.float32),
            pltpu.SemaphoreType.DMA,
            pltpu.SemaphoreType.DMA,
            pltpu.SemaphoreType.DMA,
            pltpu.SemaphoreType.DMA,
        ],
    )
    def emb(xt_hbm, tbl_hbm, out_hbm, idx_v, line_v, rows_v, obuf_v,
            gs0, gs1, os0, os1):
        wid = lax.axis_index("s") * nc + lax.axis_index("c")
        lane_iota = lax.iota(jnp.int32, nl)
        gsems = (gs0, gs1)
        osems = (os0, os1)

        def do_block(h0, n_h, b0, owaits):
            pltpu.sync_copy(xt_hbm.at[pl.ds(h0, n_h), pl.ds(b0, _BB)],
                            idx_v.at[pl.ds(0, n_h), pl.ds(0, _BB)])

            def fire(hh):
                p = hh & 1
                for k in range(_BB // nl):
                    line_v[p, pl.ds(k * nl, nl)] = (
                        idx_v[hh, pl.ds(k * nl, nl)] >> 2)
                return tuple(
                    pltpu.async_copy(
                        tbl_hbm.at[line_v.at[p, pl.ds(s * seg, seg)]],
                        rows_v.at[p, pl.ds(s * seg, seg), :], gsems[p])
                    for s in range(_NS))

            gwaits = {0: fire(0)}
            for hh in range(n_h):
                p = hh & 1
                if hh + 1 < n_h:
                    gwaits[hh + 1] = fire(hh + 1)
                for hdl in gwaits.pop(hh):
                    hdl.wait()
                if owaits[p] is not None:
                    owaits[p].wait()

                def perm_group(g, carry):
                    vec = idx_v[hh, pl.ds(g * 8, nl)]
                    remv = (vec & 3) << 5
                    for u in range(8):
                        j = g * 8 + u
                        rem32 = remv[u]
                        col = jnp.broadcast_to(j, (nl,)).astype(jnp.int32)
                        for half in range(2):
                            vals = rows_v[p, j, pl.ds(rem32 + half * nl, nl)]
                            plsc.store_scatter(
                                obuf_v.at[p],
                                [lane_iota + half * nl, col], vals)
                    return carry

                lax.fori_loop(0, _BB // 8, perm_group, 0)
                owaits[p] = pltpu.async_copy(
                    obuf_v.at[p, :, pl.ds(0, _BB)],
                    out_hbm.at[h0 + hh, :, pl.ds(b0, _BB)],
                    osems[p])
            return owaits

        def hg_body(hg, carry):
            h0 = pl.multiple_of(hg * _HG, _HG)
            owaits = [None, None]
            for bb in range(nbb):
                b0 = pl.multiple_of(wid * b_per_w + bb * _BB, _BB)
                owaits = do_block(h0, _HG, b0, owaits)
            for w in owaits:
                if w is not None:
                    w.wait()
            return carry

        lax.fori_loop(0, n_hg_full, hg_body, 0)
        if h_tail:
            owaits = [None, None]
            for bb in range(nbb):
                b0 = pl.multiple_of(wid * b_per_w + bb * _BB, _BB)
                owaits = do_block(n_hg_full * _HG, h_tail, b0, owaits)
            for w in owaits:
                if w is not None:
                    w.wait()

    return emb


def kernel(x, table):
    b, h = x.shape
    v, d = table.shape
    assert d == _D and v % 4 == 0
    tbl4 = table.reshape(v // 4, 4 * _D)
    out_t = _build(b, h, v)(x.T, tbl4)
    return out_t.transpose(2, 0, 1)
